# Initial kernel scaffold; baseline (speedup 1.0000x reference)
import functools

import jax
import jax.numpy as jnp
from jax import lax
from jax.experimental import pallas as pl
from jax.experimental.pallas import tpu as pltpu
from jax.experimental.pallas import tpu_sc as plsc

NUM_USERS = 50000
NUM_ITEMS = 50000
N = NUM_USERS + NUM_ITEMS
EMB = 32
HALF = 16
N_LAYERS = 3
E = 1600000

L = 16
CH = 128
GRP = 16
NTILES = 16
RPT = 784
EPAD = RPT * CH * NTILES
NROWS = EPAD // CH
NGRP = RPT // GRP

TRASH = N
N_ACC = 100096
ACC_PT = N_ACC // NTILES
ZB = 782
SUM_PT = N // NTILES
RB = 625

_mesh = plsc.VectorSubcoreMesh(core_axis_name="c", subcore_axis_name="s")


@functools.partial(
    pl.kernel,
    mesh=_mesh,
    out_type=(
        jax.ShapeDtypeStruct((2 * N, HALF), jnp.float32),
        jax.ShapeDtypeStruct((2 * N, HALF), jnp.float32),
        jax.ShapeDtypeStruct((2 * N, HALF), jnp.float32),
    ),
    scratch_types=[
        pltpu.VMEM_SHARED((N_ACC, HALF), jnp.float32),
        pltpu.VMEM((ZB, HALF), jnp.float32),
        pltpu.VMEM((RB, HALF), jnp.float32),
        pltpu.VMEM((RB, HALF), jnp.float32),
        pltpu.VMEM((GRP, CH), jnp.int32),
        pltpu.VMEM((GRP, CH), jnp.int32),
        pltpu.VMEM((GRP, CH), jnp.float32),
        pltpu.VMEM((CH,), jnp.int32),
        pltpu.VMEM((CH, HALF), jnp.float32),
        pltpu.SemaphoreType.DMA,
    ],
)
def _gcn(ego0, src2, dst2, w2, sum_out, ego_a, ego_b,
         accum, zbuf, rb_new, rb_sum, srcv, dstv, wvv, gidx, rows, sem):
    c = lax.axis_index("c")
    s = lax.axis_index("s")
    c_n = c * N

    def _zero(i, carry):
        zbuf[i, :] = jnp.zeros((L,), jnp.float32)
        return carry

    lax.fori_loop(0, ZB, _zero, 0)

    for k in range(SUM_PT // RB):
        r0 = c_n + s * SUM_PT + k * RB
        pltpu.sync_copy(ego0.at[pl.ds(r0, RB)], rb_new)
        pltpu.sync_copy(rb_new, sum_out.at[pl.ds(r0, RB)])

    for layer in range(N_LAYERS):
        ein = (ego0, ego_a, ego_b)[layer]
        eout = (ego_a, ego_b, ego_a)[layer]

        plsc.subcore_barrier()
        for k in range(ACC_PT // ZB):
            pltpu.sync_copy(zbuf, accum.at[pl.ds(s * ACC_PT + k * ZB, ZB)])
        plsc.subcore_barrier()

        def _group(g, carry):
            mrow = s * RPT + g * GRP
            pltpu.sync_copy(src2.at[pl.ds(mrow, GRP)], srcv)
            pltpu.sync_copy(dst2.at[pl.ds(mrow, GRP)], dstv)
            pltpu.sync_copy(w2.at[pl.ds(mrow, GRP)], wvv)

            def _chunk(j, carry2):
                for k in range(CH // L):
                    gidx[pl.ds(k * L, L)] = srcv[j, pl.ds(k * L, L)] + c_n
                pltpu.async_copy(ein.at[gidx], rows, sem).wait()
                for k in range(CH // L):
                    w16 = wvv[j, pl.ds(k * L, L)]
                    for e in range(L):
                        spl = jnp.take(
                            w16,
                            jnp.full((L,), e, jnp.int32),
                            mode=lax.GatherScatterMode.PROMISE_IN_BOUNDS,
                        )
                        r = k * L + e
                        rows[r, :] = rows[r, :] * spl
                pltpu.sync_copy(rows, accum.at[dstv.at[j]], add=True)
                return carry2

            lax.fori_loop(0, GRP, _chunk, 0)
            return carry

        lax.fori_loop(0, NGRP, _group, 0)
        plsc.subcore_barrier()

        for k in range(SUM_PT // RB):
            arow = s * SUM_PT + k * RB
            grow = c_n + arow
            pltpu.sync_copy(accum.at[pl.ds(arow, RB)], rb_new)
            pltpu.sync_copy(rb_new, eout.at[pl.ds(grow, RB)])
            pltpu.sync_copy(sum_out.at[pl.ds(grow, RB)], rb_sum)

            def _acc(i, carry):
                v = rb_sum[i, :] + rb_new[i, :]
                if layer == N_LAYERS - 1:
                    v = v * 0.25
                rb_sum[i, :] = v
                return carry

            lax.fori_loop(0, RB, _acc, 0)
            pltpu.sync_copy(rb_sum, sum_out.at[pl.ds(grow, RB)])


@jax.jit
def kernel(user_emb, item_emb, edge_weight, edge_index):
    ego0 = jnp.concatenate([user_emb, item_emb], axis=0)
    ego0t = jnp.concatenate([ego0[:, :HALF], ego0[:, HALF:]], axis=0)
    pad = EPAD - E
    src2 = jnp.concatenate(
        [edge_index[0], jnp.zeros((pad,), jnp.int32)]).reshape(NROWS, CH)
    dst2 = jnp.concatenate(
        [edge_index[1], jnp.full((pad,), TRASH, jnp.int32)]).reshape(NROWS, CH)
    w2 = jnp.concatenate(
        [edge_weight, jnp.zeros((pad,), jnp.float32)]).reshape(NROWS, CH)
    ssum, _, _ = _gcn(ego0t, src2, dst2, w2)
    out = jnp.stack([ssum[:N], ssum[N:]], axis=1).reshape(N, EMB)
    return out[:NUM_USERS], out[NUM_USERS:]


# speedup vs baseline: 6.7804x; 6.7804x over previous
import functools

import jax
import jax.numpy as jnp
from jax import lax
from jax.experimental import pallas as pl
from jax.experimental.pallas import tpu as pltpu
from jax.experimental.pallas import tpu_sc as plsc

NUM_USERS = 50000
NUM_ITEMS = 50000
N = NUM_USERS + NUM_ITEMS
EMB = 32
HALF = 16
N_LAYERS = 3
E = 1600000

L = 16
CH = 128
GRP = 16
NTILES = 16
RPT = 784
EPAD = RPT * CH * NTILES
NROWS = EPAD // CH
NGRP = RPT // GRP

TRASH = N
N_P = 100096
ACC_PT = N_P // NTILES
RB = 368
NRB = ACC_PT // RB

_mesh = plsc.VectorSubcoreMesh(core_axis_name="c", subcore_axis_name="s")


@functools.partial(
    pl.kernel,
    mesh=_mesh,
    compiler_params=pltpu.CompilerParams(use_tc_tiling_on_sc=False),
    out_type=(
        jax.ShapeDtypeStruct((2 * N_P, HALF), jnp.float32),
        jax.ShapeDtypeStruct((2 * N_P, HALF), jnp.float32),
        jax.ShapeDtypeStruct((2 * N_P, HALF), jnp.float32),
    ),
    scratch_types=[
        pltpu.VMEM_SHARED((N_P, HALF), jnp.float32),
        pltpu.VMEM((RB, HALF), jnp.float32),
        pltpu.VMEM((RB, HALF), jnp.float32),
        pltpu.VMEM((RB, HALF), jnp.float32),
        pltpu.VMEM((GRP, CH), jnp.int32),
        pltpu.VMEM((GRP, CH), jnp.int32),
        pltpu.VMEM((GRP, CH), jnp.float32),
        pltpu.VMEM((CH,), jnp.int32),
        pltpu.VMEM((CH, HALF), jnp.float32),
        pltpu.SemaphoreType.DMA,
    ],
)
def _gcn(ego0, src2, dst2, w2, sum_out, ego_a, ego_b,
         accum, zbuf, rb_new, rb_sum, srcv, dstv, wvv, gidx, rows, sem):
    c = lax.axis_index("c")
    s = lax.axis_index("s")
    c_n = c * N_P

    def _zero(i, carry):
        zbuf[i, :] = jnp.zeros((L,), jnp.float32)
        return carry

    lax.fori_loop(0, RB, _zero, 0)

    for k in range(NRB):
        r0 = c_n + s * ACC_PT + k * RB
        pltpu.sync_copy(ego0.at[pl.ds(r0, RB)], rb_new)
        pltpu.sync_copy(rb_new, sum_out.at[pl.ds(r0, RB)])

    for layer in range(N_LAYERS):
        ein = (ego0, ego_a, ego_b)[layer]
        eout = (ego_a, ego_b, ego_a)[layer]

        plsc.subcore_barrier()
        for k in range(NRB):
            pltpu.sync_copy(zbuf, accum.at[pl.ds(s * ACC_PT + k * RB, RB)])
        plsc.subcore_barrier()

        def _group(g, carry):
            mrow = s * RPT + g * GRP
            pltpu.sync_copy(src2.at[pl.ds(mrow, GRP)], srcv)
            pltpu.sync_copy(dst2.at[pl.ds(mrow, GRP)], dstv)
            pltpu.sync_copy(w2.at[pl.ds(mrow, GRP)], wvv)

            def _chunk(j, carry2):
                for k in range(CH // L):
                    gidx[pl.ds(k * L, L)] = srcv[j, pl.ds(k * L, L)] + c_n
                pltpu.async_copy(ein.at[gidx], rows, sem).wait()
                for k in range(CH // L):
                    w16 = wvv[j, pl.ds(k * L, L)]
                    for e in range(L):
                        spl = w16.at[jnp.full((L,), e, jnp.int32)].get(
                            mode="promise_in_bounds")
                        r = k * L + e
                        rows[r, :] = rows[r, :] * spl
                pltpu.sync_copy(rows, accum.at[dstv.at[j]], add=True)
                return carry2

            lax.fori_loop(0, GRP, _chunk, 0)
            return carry

        lax.fori_loop(0, NGRP, _group, 0)
        plsc.subcore_barrier()

        for k in range(NRB):
            arow = s * ACC_PT + k * RB
            grow = c_n + arow
            pltpu.sync_copy(accum.at[pl.ds(arow, RB)], rb_new)
            pltpu.sync_copy(rb_new, eout.at[pl.ds(grow, RB)])
            pltpu.sync_copy(sum_out.at[pl.ds(grow, RB)], rb_sum)

            def _acc(i, carry):
                v = rb_sum[i, :] + rb_new[i, :]
                if layer == N_LAYERS - 1:
                    v = v * 0.25
                rb_sum[i, :] = v
                return carry

            lax.fori_loop(0, RB, _acc, 0)
            pltpu.sync_copy(rb_sum, sum_out.at[pl.ds(grow, RB)])


@jax.jit
def kernel(user_emb, item_emb, edge_weight, edge_index):
    ego0 = jnp.concatenate(
        [user_emb, item_emb, jnp.zeros((N_P - N, EMB), jnp.float32)], axis=0)
    ego0t = jnp.concatenate([ego0[:, :HALF], ego0[:, HALF:]], axis=0)
    pad = EPAD - E
    src2 = jnp.concatenate(
        [edge_index[0], jnp.zeros((pad,), jnp.int32)]).reshape(NROWS, CH)
    dst2 = jnp.concatenate(
        [edge_index[1], jnp.full((pad,), TRASH, jnp.int32)]).reshape(NROWS, CH)
    w2 = jnp.concatenate(
        [edge_weight, jnp.zeros((pad,), jnp.float32)]).reshape(NROWS, CH)
    ssum, _, _ = _gcn(ego0t, src2, dst2, w2)
    out = jnp.stack([ssum[:N], ssum[N_P:N_P + N]], axis=1).reshape(N, EMB)
    return out[:NUM_USERS], out[NUM_USERS:]
